# baseline (speedup 1.0000x reference)
import jax
import jax.numpy as jnp
from jax import lax
from jax.experimental import pallas as pl
from jax.experimental.pallas import tpu as pltpu
from jax.experimental.pallas import tpu_sc as plsc

NUM_CORES = 2
NUM_SUBCORES = 16
NUM_WORKERS = NUM_CORES * NUM_SUBCORES
LANES = 16
BATCH = 16384
PER_WORKER = BATCH // NUM_WORKERS
NCHUNK = 4
CHUNK = PER_WORKER // NCHUNK


def _sc_body(idx_hbm, loss_hbm, sigma_hbm, sl_hbm, sel_hbm,
             idx_v, loss_v, sel_v, sl_v, sem_l, sem_o, *sem_g):
    wid = lax.axis_index("s") * NUM_CORES + lax.axis_index("c")
    base = wid * PER_WORKER

    loss_cp = pltpu.async_copy(loss_hbm.at[pl.ds(base, PER_WORKER)], loss_v,
                               sem_l)
    pltpu.sync_copy(idx_hbm.at[pl.ds(base, PER_WORKER)], idx_v)
    gathers = [
        pltpu.async_copy(sigma_hbm.at[idx_v.at[pl.ds(c * CHUNK, CHUNK)]],
                         sel_v.at[pl.ds(c * CHUNK, CHUNK)], sem_g[c])
        for c in range(NCHUNK)
    ]
    loss_cp.wait()

    outs = []
    for c in range(NCHUNK):
        gathers[c].wait()
        for c0 in range(c * CHUNK, (c + 1) * CHUNK, LANES):
            sl_v[pl.ds(c0, LANES)] = (
                sel_v[pl.ds(c0, LANES)] * loss_v[pl.ds(c0, LANES)]
            )
        outs.append(pltpu.async_copy(
            sl_v.at[pl.ds(c * CHUNK, CHUNK)],
            sl_hbm.at[pl.ds(base + c * CHUNK, CHUNK)], sem_o))
        outs.append(pltpu.async_copy(
            sel_v.at[pl.ds(c * CHUNK, CHUNK)],
            sel_hbm.at[pl.ds(base + c * CHUNK, CHUNK)], sem_o))
    for o in outs:
        o.wait()


def kernel(loss, basin_idx, sigma):
    idx = basin_idx.astype(jnp.int32)

    mesh = plsc.VectorSubcoreMesh(
        core_axis_name="c", subcore_axis_name="s",
        num_cores=NUM_CORES, num_subcores=NUM_SUBCORES,
    )
    out_type = (
        jax.ShapeDtypeStruct((BATCH,), jnp.float32),
        jax.ShapeDtypeStruct((BATCH,), jnp.float32),
    )
    scratch = [
        pltpu.VMEM((PER_WORKER,), jnp.int32),
        pltpu.VMEM((PER_WORKER,), jnp.float32),
        pltpu.VMEM((PER_WORKER,), jnp.float32),
        pltpu.VMEM((PER_WORKER,), jnp.float32),
        pltpu.SemaphoreType.DMA,
        pltpu.SemaphoreType.DMA,
    ] + [pltpu.SemaphoreType.DMA] * NCHUNK
    superloss, sel = pl.kernel(
        _sc_body, out_type=out_type, mesh=mesh, scratch_types=scratch,
    )(idx, loss, sigma)
    return superloss, sel
